# baseline (speedup 1.0000x reference)
import functools

import jax
import jax.numpy as jnp
from jax.experimental import pallas as pl
from jax.experimental.pallas import tpu as pltpu

RADIUS = 3
T = 2
IN_FEAT = 39
BOND_DIM = 10
FP = 64
L = 64
NB = 6
NBL = NB * L
MB = 16

_NEG = -9e8
_SLOPE = 0.01


def _leaky(x):
    return jnp.where(x > 0, x, _SLOPE * x)


def _elu(x):
    return jnp.where(x > 0, x, jnp.exp(jnp.minimum(x, 0.0)) - 1.0)


def _dot(a, b):
    return jnp.dot(a, b, preferred_element_type=jnp.float32)


def _split2(x):
    hi = x.astype(jnp.bfloat16).astype(jnp.float32)
    return hi, x - hi


def _gdot(a, b):
    return jnp.dot(a, b, preferred_element_type=jnp.float32,
                   precision=jax.lax.Precision.HIGHEST)


def _fused_kernel(
    atom_ref, bond_ref, adl_ref, bdl_ref, amask_ref,
    w_atom_ref, b_atom_ref, w_nb_a_ref, w_nb_b_ref, b_nb_ref,
    gru_wih_ref, gru_whh_ref, gru_bih_ref, gru_bhh_ref,
    w_align_ref, b_align_ref, w_attend_ref, b_attend_ref,
    mgru_wih_ref, mgru_whh_ref, mgru_bih_ref, mgru_bhh_ref,
    w_malign_ref, b_malign_ref, w_mattend_ref, b_mattend_ref,
    w_m1_ref, w_m2_ref, b_metric_ref, w_out_ref, b_out_ref,
    out_ref,
):
    f32 = jnp.float32
    atom = atom_ref[:]
    bond = bond_ref[:]
    adl = adl_ref[:]
    bdl = bdl_ref[:]
    am3 = amask_ref[:]

    iota = jax.lax.broadcasted_iota(jnp.int32, (NBL, L), 1)
    oh_a = [(adl[m] == iota).astype(f32) for m in range(MB)]
    oh_b = [(bdl[m] == iota).astype(f32) for m in range(MB)]

    attend_mask = (adl != L - 1).astype(f32)
    smask = jnp.where(adl == L - 1, _NEG, 0.0).astype(f32)

    atom_flat = atom.reshape(MB * L, IN_FEAT)
    af = _leaky(_dot(atom_flat, w_atom_ref[:]) + b_atom_ref[:])
    af3 = af.reshape(MB, L, FP)

    atom_hi, atom_lo = _split2(atom)
    bond_hi, bond_lo = _split2(bond)
    ga = jnp.concatenate(
        [_dot(oh_a[m], atom_hi[m]) + _dot(oh_a[m], atom_lo[m])
         for m in range(MB)], axis=0)
    gb = jnp.concatenate(
        [_dot(oh_b[m], bond_hi[m]) + _dot(oh_b[m], bond_lo[m])
         for m in range(MB)], axis=0)
    nbf = _leaky(_dot(ga, w_nb_a_ref[:]) + _dot(gb, w_nb_b_ref[:])
                 + b_nb_ref[:])

    def attention(act3, nb_flat, s2, d):
        w1 = w_align_ref[2 * d:2 * d + 1, :].reshape(1, 1, FP)
        bal = b_align_ref[d:d + 1, :].reshape(1, 1, 1)
        s1 = jnp.sum(act3 * w1, axis=-1, keepdims=True)
        nt = _dot(nb_flat, w_attend_ref[d]) + b_attend_ref[d:d + 1, :]
        nt3 = nt.reshape(MB, NBL, FP)
        sj = [
            _leaky(s1 + s2[:, j * L:(j + 1) * L] + bal)
            + smask[:, j * L:(j + 1) * L]
            for j in range(NB)
        ]
        mx = functools.reduce(jnp.maximum, sj)
        ej = [jnp.exp(s - mx) for s in sj]
        z = functools.reduce(jnp.add, ej)
        ctx = functools.reduce(jnp.add, [
            (ej[j] / z) * attend_mask[:, j * L:(j + 1) * L]
            * nt3[:, j * L:(j + 1) * L, :]
            for j in range(NB)
        ])
        return _elu(ctx)

    def gru(x, h, wih_ref, whh_ref, bih_ref, bhh_ref, base):
        gi_r = _dot(x, wih_ref[base + 0]) + bih_ref[base + 0:base + 1, :]
        gi_z = _dot(x, wih_ref[base + 1]) + bih_ref[base + 1:base + 2, :]
        gi_n = _dot(x, wih_ref[base + 2]) + bih_ref[base + 2:base + 3, :]
        gh_r = _dot(h, whh_ref[base + 0]) + bhh_ref[base + 0:base + 1, :]
        gh_z = _dot(h, whh_ref[base + 1]) + bhh_ref[base + 1:base + 2, :]
        gh_n = _dot(h, whh_ref[base + 2]) + bhh_ref[base + 2:base + 3, :]
        r = jax.nn.sigmoid(gi_r + gh_r)
        z = jax.nn.sigmoid(gi_z + gh_z)
        n = jnp.tanh(gi_n + r * gh_n)
        return (1.0 - z) * n + z * h

    w2_0 = w_align_ref[1:2, :]
    s2_0 = jnp.sum(nbf * w2_0, axis=-1,
                   keepdims=True).reshape(MB, NBL, 1)
    ctx3 = attention(af3, nbf, s2_0, 0)
    h = gru(ctx3.reshape(MB * L, FP), af,
            gru_wih_ref, gru_whh_ref, gru_bih_ref, gru_bhh_ref, 0)
    act3 = jnp.maximum(h, 0.0).reshape(MB, L, FP)

    for d in range(1, RADIUS):
        act_hi, act_lo = _split2(act3)
        nbg = jnp.concatenate(
            [_dot(oh_a[m], act_hi[m]) + _dot(oh_a[m], act_lo[m])
             for m in range(MB)], axis=0)
        w2d = w_align_ref[2 * d + 1:2 * d + 2, :].reshape(1, 1, FP)
        p = jnp.sum(act3 * w2d, axis=-1, keepdims=True)
        s2 = jnp.concatenate([_gdot(oh_a[m], p[m]) for m in range(MB)],
                             axis=0).reshape(MB, NBL, 1)
        ctx3 = attention(act3, nbg, s2, d)
        h = gru(ctx3.reshape(MB * L, FP), h,
                gru_wih_ref, gru_whh_ref, gru_bih_ref, gru_bhh_ref, 3 * d)
        act3 = jnp.maximum(h, 0.0).reshape(MB, L, FP)

    mol = jnp.sum(act3 * am3, axis=1)
    act_mol = jnp.maximum(mol, 0.0)
    msm = jnp.where(am3 == 0.0, _NEG, 0.0).astype(f32)

    w1m = w_malign_ref[0:1, :]
    w2m = w_malign_ref[1:2, :].reshape(1, 1, FP)
    bmal = b_malign_ref[0, 0]
    s_atom = jnp.sum(act3 * w2m, axis=-1, keepdims=True)
    at = _dot(act3.reshape(MB * L, FP), w_mattend_ref[:]) + b_mattend_ref[:]
    at3 = at.reshape(MB, L, FP)

    for _ in range(T):
        s_mol = jnp.sum(act_mol * w1m, axis=-1,
                        keepdims=True).reshape(MB, 1, 1)
        ms = _leaky(s_mol + s_atom + bmal) + msm
        mx = jnp.max(ms, axis=1, keepdims=True)
        e = jnp.exp(ms - mx)
        z = jnp.sum(e, axis=1, keepdims=True)
        maw = (e / z) * am3
        mc = _elu(jnp.sum(maw * at3, axis=1))
        mol = gru(mc, mol, mgru_wih_ref, mgru_whh_ref,
                  mgru_bih_ref, mgru_bhh_ref, 0)
        act_mol = jnp.maximum(mol, 0.0)

    d_val = float(RADIUS - 2) if RADIUS > 1 else 0.0
    m1 = (_dot(mol, w_m1_ref[:]) + _dot(mol + d_val, w_m2_ref[:])
          + b_metric_ref[:])
    out_ref[:] = _dot(m1, w_out_ref[:]) + b_out_ref[:]


def kernel(atom_list, bond_list, atom_degree_list, bond_degree_list,
           atom_mask, W_atom, b_atom, W_nb, b_nb, gru_wih, gru_whh,
           gru_bih, gru_bhh, W_align, b_align, W_attend, b_attend,
           mgru_wih, mgru_whh, mgru_bih, mgru_bhh, W_malign, b_malign,
           W_mattend, b_mattend, W_metric, b_metric, W_out, b_out):
    B = atom_list.shape[0]
    f32 = jnp.float32

    adl = jnp.transpose(atom_degree_list, (0, 2, 1)).reshape(B, NBL, 1)
    bdl = jnp.transpose(bond_degree_list, (0, 2, 1)).reshape(B, NBL, 1)
    adl = adl.astype(jnp.int32)
    bdl = bdl.astype(jnp.int32)
    amask = atom_mask.reshape(B, L, 1).astype(f32)

    w_atom_t = W_atom.T
    w_nb_a_t = W_nb[:, :IN_FEAT].T
    w_nb_b_t = W_nb[:, IN_FEAT:].T
    gru_wih_t = gru_wih.reshape(RADIUS * 3, FP, FP).transpose(0, 2, 1)
    gru_whh_t = gru_whh.reshape(RADIUS * 3, FP, FP).transpose(0, 2, 1)
    gru_bih2 = gru_bih.reshape(RADIUS * 3, FP)
    gru_bhh2 = gru_bhh.reshape(RADIUS * 3, FP)
    w_align2 = W_align.reshape(RADIUS * 2, FP)
    b_align2 = b_align.reshape(RADIUS, 1)
    w_attend_t = W_attend.transpose(0, 2, 1)
    b_attend2 = b_attend.reshape(RADIUS, FP)
    mgru_wih_t = mgru_wih.reshape(3, FP, FP).transpose(0, 2, 1)
    mgru_whh_t = mgru_whh.reshape(3, FP, FP).transpose(0, 2, 1)
    mgru_bih2 = mgru_bih.reshape(3, FP)
    mgru_bhh2 = mgru_bhh.reshape(3, FP)
    w_malign2 = W_malign.reshape(2, FP)
    b_malign2 = b_malign.reshape(1, 1)
    w_mattend_t = W_mattend.T
    b_mattend2 = b_mattend.reshape(1, FP)
    w_m1_t = W_metric[:, :FP].T
    w_m2_t = W_metric[:, FP:].T
    b_metric2 = b_metric.reshape(1, FP)
    w_out_t = W_out.T
    b_out2 = b_out.reshape(1, 1)
    b_atom2 = b_atom.reshape(1, FP)
    b_nb2 = b_nb.reshape(1, FP)

    grid = (B // MB,)

    def dspec(block):
        nd = len(block)
        return pl.BlockSpec(block, lambda i, nd=nd: (i,) + (0,) * (nd - 1))

    def wspec(shape):
        nd = len(shape)
        return pl.BlockSpec(shape, lambda i, nd=nd: (0,) * nd)

    args = (
        atom_list, bond_list, adl, bdl, amask,
        w_atom_t, b_atom2, w_nb_a_t, w_nb_b_t, b_nb2,
        gru_wih_t, gru_whh_t, gru_bih2, gru_bhh2,
        w_align2, b_align2, w_attend_t, b_attend2,
        mgru_wih_t, mgru_whh_t, mgru_bih2, mgru_bhh2,
        w_malign2, b_malign2, w_mattend_t, b_mattend2,
        w_m1_t, w_m2_t, b_metric2, w_out_t, b_out2,
    )
    in_specs = [
        dspec((MB, L, IN_FEAT)), dspec((MB, L, BOND_DIM)),
        dspec((MB, NBL, 1)), dspec((MB, NBL, 1)), dspec((MB, L, 1)),
    ] + [wspec(a.shape) for a in args[5:]]

    out = pl.pallas_call(
        _fused_kernel,
        grid=grid,
        in_specs=in_specs,
        out_specs=pl.BlockSpec((MB, 1), lambda i: (i, 0)),
        out_shape=jax.ShapeDtypeStruct((B, 1), f32),
        compiler_params=pltpu.CompilerParams(
            dimension_semantics=("arbitrary",),
        ),
    )(*args)
    return out


# speedup vs baseline: 1.8660x; 1.8660x over previous
import jax
import jax.numpy as jnp
from jax.experimental import pallas as pl
from jax.experimental.pallas import tpu as pltpu

RADIUS = 3
T = 2
IN_FEAT = 39
BOND_DIM = 10
FP = 64
L = 64
NB = 6
NBL = NB * L
MB = 16

_NEG = -9e8
_SLOPE = 0.01


def _leaky(x):
    return jnp.where(x > 0, x, _SLOPE * x)


def _elu(x):
    return jnp.where(x > 0, x, jnp.exp(jnp.minimum(x, 0.0)) - 1.0)


def _dot(a, b):
    return jnp.dot(a, b, preferred_element_type=jnp.float32)


def _hdot(a, b):
    return jnp.dot(a, b, preferred_element_type=jnp.float32,
                   precision=jax.lax.Precision.HIGHEST)


def _split2(x):
    hi = x.astype(jnp.bfloat16).astype(jnp.float32)
    return hi, x - hi


def _fused_kernel(
    atom_ref, bond_ref, adl_ref, bdl_ref, amask_ref,
    w_atom_ref, b_atom_ref, w_nb_a_ref, w_nb_b_ref, b_nb_ref,
    gru_wih_ref, gru_whh_ref, gru_bih_ref, gru_bhh_ref,
    w_align_ref, b_align_ref, w_attend_ref, b_attend_ref,
    mgru_wih_ref, mgru_whh_ref, mgru_bih_ref, mgru_bhh_ref,
    w_malign_ref, b_malign_ref, w_mattend_ref, b_mattend_ref,
    w_m1_ref, w_m2_ref, b_metric_ref, w_out_ref, b_out_ref,
    out_ref,
):
    f32 = jnp.float32
    xa = atom_ref[:]
    xb = bond_ref[:]
    adl3 = adl_ref[:]
    bdl3 = bdl_ref[:]
    am3 = amask_ref[:]

    iota_s = jax.lax.broadcasted_iota(jnp.int32, (L, NBL), 0)
    oh_a = [(adl3[m] == iota_s).astype(f32) for m in range(MB)]
    oh_b = [(bdl3[m] == iota_s).astype(f32) for m in range(MB)]

    attend3 = (adl3 != L - 1).astype(f32)
    smask3 = jnp.where(adl3 == L - 1, _NEG, 0.0).astype(f32)

    def msl(m):
        return slice(m * L, (m + 1) * L)

    def jsl(j):
        return slice(j * L, (j + 1) * L)

    def gather(hi, lo, oh):
        return jnp.concatenate(
            [_dot(hi[:, msl(m)], oh[m]) + _dot(lo[:, msl(m)], oh[m])
             for m in range(MB)], axis=1)

    af = _leaky(_dot(w_atom_ref[:], xa) + b_atom_ref[:])

    xa_hi, xa_lo = _split2(xa)
    xb_hi, xb_lo = _split2(xb)
    ga = gather(xa_hi, xa_lo, oh_a)
    gb = gather(xb_hi, xb_lo, oh_b)
    nbf = _leaky(_dot(w_nb_a_ref[:], ga) + _dot(w_nb_b_ref[:], gb)
                 + b_nb_ref[:])

    def attention(act, nb, s23, d):
        w1 = w_align_ref[2 * d:2 * d + 1, :]
        bal = b_align_ref[d:d + 1, :].reshape(1, 1, 1)
        s1row = _hdot(w1, act)
        s13 = jnp.stack([s1row[:, msl(m)] for m in range(MB)])
        nt = _dot(w_attend_ref[d], nb) + b_attend_ref[d]
        sj = [
            _leaky(s13 + s23[:, :, jsl(j)] + bal) + smask3[:, :, jsl(j)]
            for j in range(NB)
        ]
        mx = sj[0]
        for s in sj[1:]:
            mx = jnp.maximum(mx, s)
        ej = [jnp.exp(s - mx) for s in sj]
        z = ej[0]
        for e in ej[1:]:
            z = z + e
        aw3 = jnp.concatenate(
            [(ej[j] / z) * attend3[:, :, jsl(j)] for j in range(NB)],
            axis=2)
        ctxs = []
        for m in range(MB):
            ws = nt[:, m * NBL:(m + 1) * NBL] * aw3[m]
            acc = ws[:, jsl(0)]
            for j in range(1, NB):
                acc = acc + ws[:, jsl(j)]
            ctxs.append(acc)
        return _elu(jnp.concatenate(ctxs, axis=1))

    def gru(x, h, wih_ref, whh_ref, bih_ref, bhh_ref, base):
        gi_r = _dot(wih_ref[base + 0], x) + bih_ref[base + 0]
        gi_z = _dot(wih_ref[base + 1], x) + bih_ref[base + 1]
        gi_n = _dot(wih_ref[base + 2], x) + bih_ref[base + 2]
        gh_r = _dot(whh_ref[base + 0], h) + bhh_ref[base + 0]
        gh_z = _dot(whh_ref[base + 1], h) + bhh_ref[base + 1]
        gh_n = _dot(whh_ref[base + 2], h) + bhh_ref[base + 2]
        r = jax.nn.sigmoid(gi_r + gh_r)
        z = jax.nn.sigmoid(gi_z + gh_z)
        n = jnp.tanh(gi_n + r * gh_n)
        return (1.0 - z) * n + z * h

    w2_0 = w_align_ref[1:2, :]
    s2row = _hdot(w2_0, nbf)
    s23 = jnp.stack([s2row[:, m * NBL:(m + 1) * NBL] for m in range(MB)])
    ctx = attention(af, nbf, s23, 0)
    h = gru(ctx, af, gru_wih_ref, gru_whh_ref, gru_bih_ref, gru_bhh_ref, 0)
    act = jnp.maximum(h, 0.0)

    for d in range(1, RADIUS):
        act_hi, act_lo = _split2(act)
        nbg = gather(act_hi, act_lo, oh_a)
        w2 = w_align_ref[2 * d + 1:2 * d + 2, :]
        prow = _hdot(w2, act)
        s23 = jnp.stack(
            [_hdot(prow[:, msl(m)], oh_a[m]) for m in range(MB)])
        ctx = attention(act, nbg, s23, d)
        h = gru(ctx, h, gru_wih_ref, gru_whh_ref, gru_bih_ref,
                gru_bhh_ref, 3 * d)
        act = jnp.maximum(h, 0.0)

    amrow = jnp.concatenate([am3[m] for m in range(MB)], axis=1)
    masked = act * amrow
    mol = jnp.concatenate(
        [jnp.sum(masked[:, msl(m)], axis=1, keepdims=True)
         for m in range(MB)], axis=1)
    act_mol = jnp.maximum(mol, 0.0)
    msmrow = jnp.where(amrow == 0.0, _NEG, 0.0).astype(f32)
    msm3 = jnp.stack([msmrow[:, msl(m)] for m in range(MB)])
    am3s = jnp.stack([amrow[:, msl(m)] for m in range(MB)])

    w1m = w_malign_ref[0:1, :]
    w2m = w_malign_ref[1:2, :]
    bmal = b_malign_ref[0, 0]
    s_atomrow = _hdot(w2m, act)
    s_atom3 = jnp.stack([s_atomrow[:, msl(m)] for m in range(MB)])
    at = _dot(w_mattend_ref[:], act) + b_mattend_ref[:]

    for _ in range(T):
        s_molrow = _hdot(w1m, act_mol)
        ms3 = jnp.stack(
            [_leaky(s_molrow[:, m:m + 1] + s_atom3[m] + bmal) + msm3[m]
             for m in range(MB)])
        mx = jnp.max(ms3, axis=2, keepdims=True)
        e = jnp.exp(ms3 - mx)
        z = jnp.sum(e, axis=2, keepdims=True)
        maw3 = (e / z) * am3s
        mc = jnp.concatenate(
            [jnp.sum(at[:, msl(m)] * maw3[m], axis=1, keepdims=True)
             for m in range(MB)], axis=1)
        mc = _elu(mc)
        mol = gru(mc, mol, mgru_wih_ref, mgru_whh_ref,
                  mgru_bih_ref, mgru_bhh_ref, 0)
        act_mol = jnp.maximum(mol, 0.0)

    d_val = float(RADIUS - 2) if RADIUS > 1 else 0.0
    m1 = (_dot(w_m1_ref[:], mol) + _dot(w_m2_ref[:], mol + d_val)
          + b_metric_ref[:])
    outrow = _hdot(w_out_ref[:], m1) + b_out_ref[:]
    out_ref[:] = outrow.reshape(1, 1, MB)


def kernel(atom_list, bond_list, atom_degree_list, bond_degree_list,
           atom_mask, W_atom, b_atom, W_nb, b_nb, gru_wih, gru_whh,
           gru_bih, gru_bhh, W_align, b_align, W_attend, b_attend,
           mgru_wih, mgru_whh, mgru_bih, mgru_bhh, W_malign, b_malign,
           W_mattend, b_mattend, W_metric, b_metric, W_out, b_out):
    B = atom_list.shape[0]
    f32 = jnp.float32

    atom_t = jnp.transpose(atom_list, (2, 0, 1)).reshape(IN_FEAT, B * L)
    bond_t = jnp.transpose(bond_list, (2, 0, 1)).reshape(BOND_DIM, B * L)
    adl = jnp.transpose(atom_degree_list, (0, 2, 1)).reshape(B, 1, NBL)
    bdl = jnp.transpose(bond_degree_list, (0, 2, 1)).reshape(B, 1, NBL)
    adl = adl.astype(jnp.int32)
    bdl = bdl.astype(jnp.int32)
    amask = atom_mask.reshape(B, 1, L).astype(f32)

    gru_wih2 = gru_wih.reshape(RADIUS * 3, FP, FP)
    gru_whh2 = gru_whh.reshape(RADIUS * 3, FP, FP)
    gru_bih2 = gru_bih.reshape(RADIUS * 3, FP, 1)
    gru_bhh2 = gru_bhh.reshape(RADIUS * 3, FP, 1)
    w_align2 = W_align.reshape(RADIUS * 2, FP)
    b_align2 = b_align.reshape(RADIUS, 1)
    b_attend2 = b_attend.reshape(RADIUS, FP, 1)
    mgru_wih2 = mgru_wih.reshape(3, FP, FP)
    mgru_whh2 = mgru_whh.reshape(3, FP, FP)
    mgru_bih2 = mgru_bih.reshape(3, FP, 1)
    mgru_bhh2 = mgru_bhh.reshape(3, FP, 1)
    w_malign2 = W_malign.reshape(2, FP)
    b_malign2 = b_malign.reshape(1, 1)
    b_mattend2 = b_mattend.reshape(FP, 1)
    w_m1 = W_metric[:, :FP]
    w_m2 = W_metric[:, FP:]
    b_metric2 = b_metric.reshape(FP, 1)
    b_out2 = b_out.reshape(1, 1)
    b_atom2 = b_atom.reshape(FP, 1)
    b_nb2 = b_nb.reshape(FP, 1)
    w_nb_a = W_nb[:, :IN_FEAT]
    w_nb_b = W_nb[:, IN_FEAT:]

    grid = (B // MB,)

    args = (
        atom_t, bond_t, adl, bdl, amask,
        W_atom, b_atom2, w_nb_a, w_nb_b, b_nb2,
        gru_wih2, gru_whh2, gru_bih2, gru_bhh2,
        w_align2, b_align2, W_attend, b_attend2,
        mgru_wih2, mgru_whh2, mgru_bih2, mgru_bhh2,
        w_malign2, b_malign2, W_mattend, b_mattend2,
        w_m1, w_m2, b_metric2, W_out, b_out2,
    )

    def wspec(shape):
        nd = len(shape)
        return pl.BlockSpec(shape, lambda i, nd=nd: (0,) * nd)

    in_specs = [
        pl.BlockSpec((IN_FEAT, MB * L), lambda i: (0, i)),
        pl.BlockSpec((BOND_DIM, MB * L), lambda i: (0, i)),
        pl.BlockSpec((MB, 1, NBL), lambda i: (i, 0, 0)),
        pl.BlockSpec((MB, 1, NBL), lambda i: (i, 0, 0)),
        pl.BlockSpec((MB, 1, L), lambda i: (i, 0, 0)),
    ] + [wspec(a.shape) for a in args[5:]]

    out = pl.pallas_call(
        _fused_kernel,
        grid=grid,
        in_specs=in_specs,
        out_specs=pl.BlockSpec((1, 1, MB), lambda i: (i, 0, 0)),
        out_shape=jax.ShapeDtypeStruct((B // MB, 1, MB), f32),
        compiler_params=pltpu.CompilerParams(
            dimension_semantics=("arbitrary",),
        ),
    )(*args)
    return out.reshape(B, 1)


# speedup vs baseline: 1.9780x; 1.0600x over previous
import jax
import jax.numpy as jnp
from jax.experimental import pallas as pl
from jax.experimental.pallas import tpu as pltpu

RADIUS = 3
T = 2
IN_FEAT = 39
BOND_DIM = 10
FP = 64
L = 64
NB = 6
NBL = NB * L
MB = 16

_NEG = -9e8
_SLOPE = 0.01


def _leaky(x):
    return jnp.where(x > 0, x, _SLOPE * x)


def _elu(x):
    return jnp.where(x > 0, x, jnp.exp(jnp.minimum(x, 0.0)) - 1.0)


def _dot(a, b):
    return jnp.dot(a, b, preferred_element_type=jnp.float32)


def _hdot(a, b):
    return jnp.dot(a, b, preferred_element_type=jnp.float32,
                   precision=jax.lax.Precision.HIGHEST)


def _split2(x):
    hi = x.astype(jnp.bfloat16).astype(jnp.float32)
    return hi, x - hi


def _fused_kernel(
    atom_ref, bond_ref, adl_ref, bdl_ref, amask_ref,
    w_atom_ref, b_atom_ref, w_nb_a_ref, w_nb_b_ref, b_nb_ref,
    gru_wih_ref, gru_whh_ref, gru_bih_ref, gru_bhh_ref,
    w_align_ref, b_align_ref, w_attend_ref, b_attend_ref,
    mgru_wih_ref, mgru_whh_ref, mgru_bih_ref, mgru_bhh_ref,
    w_malign_ref, b_malign_ref, w_mattend_ref, b_mattend_ref,
    w_m1_ref, w_m2_ref, b_metric_ref, w_out_ref, b_out_ref,
    out_ref,
):
    f32 = jnp.float32
    xa = atom_ref[:]
    xb = bond_ref[:]
    adl3 = adl_ref[:]
    bdl3 = bdl_ref[:]
    am3 = amask_ref[:]

    iota_s = jax.lax.broadcasted_iota(jnp.int32, (L, NBL), 0)
    oh_a = [(adl3[m] == iota_s).astype(f32) for m in range(MB)]
    oh_b = [(bdl3[m] == iota_s).astype(f32) for m in range(MB)]

    attend3 = (adl3 != L - 1).astype(f32)
    smask3 = jnp.where(adl3 == L - 1, _NEG, 0.0).astype(f32)

    def msl(m):
        return slice(m * L, (m + 1) * L)

    def jsl(j):
        return slice(j * L, (j + 1) * L)

    def gather(hi, lo, oh):
        return jnp.concatenate(
            [_dot(hi[:, msl(m)], oh[m]) + _dot(lo[:, msl(m)], oh[m])
             for m in range(MB)], axis=1)

    af = _leaky(_dot(w_atom_ref[:], xa) + b_atom_ref[:])

    xa_hi, xa_lo = _split2(xa)
    xb_hi, xb_lo = _split2(xb)
    ga = gather(xa_hi, xa_lo, oh_a)
    gb = gather(xb_hi, xb_lo, oh_b)
    nbf = _leaky(_dot(w_nb_a_ref[:], ga) + _dot(w_nb_b_ref[:], gb)
                 + b_nb_ref[:])

    def attention(act, nb, d):
        w1 = w_align_ref[2 * d:2 * d + 1, :]
        w2 = w_align_ref[2 * d + 1:2 * d + 2, :]
        bal = b_align_ref[d:d + 1, :].reshape(1, 1, 1)
        s2row = _hdot(w2, nb)
        s23 = jnp.stack(
            [s2row[:, m * NBL:(m + 1) * NBL] for m in range(MB)])
        s1row = _hdot(w1, act)
        s13 = jnp.stack([s1row[:, msl(m)] for m in range(MB)])
        nt = _dot(w_attend_ref[d], nb) + b_attend_ref[d]
        sj = [
            _leaky(s13 + s23[:, :, jsl(j)] + bal) + smask3[:, :, jsl(j)]
            for j in range(NB)
        ]
        mx = sj[0]
        for s in sj[1:]:
            mx = jnp.maximum(mx, s)
        ej = [jnp.exp(s - mx) for s in sj]
        z = ej[0]
        for e in ej[1:]:
            z = z + e
        aw3 = jnp.concatenate(
            [(ej[j] / z) * attend3[:, :, jsl(j)] for j in range(NB)],
            axis=2)
        ctxs = []
        for m in range(MB):
            ws = nt[:, m * NBL:(m + 1) * NBL] * aw3[m]
            acc = ws[:, jsl(0)]
            for j in range(1, NB):
                acc = acc + ws[:, jsl(j)]
            ctxs.append(acc)
        return _elu(jnp.concatenate(ctxs, axis=1))

    def gru(x, h, wih_ref, whh_ref, bih_ref, bhh_ref, base):
        gi_r = _dot(wih_ref[base + 0], x) + bih_ref[base + 0]
        gi_z = _dot(wih_ref[base + 1], x) + bih_ref[base + 1]
        gi_n = _dot(wih_ref[base + 2], x) + bih_ref[base + 2]
        gh_r = _dot(whh_ref[base + 0], h) + bhh_ref[base + 0]
        gh_z = _dot(whh_ref[base + 1], h) + bhh_ref[base + 1]
        gh_n = _dot(whh_ref[base + 2], h) + bhh_ref[base + 2]
        r = jax.nn.sigmoid(gi_r + gh_r)
        z = jax.nn.sigmoid(gi_z + gh_z)
        n = jnp.tanh(gi_n + r * gh_n)
        return (1.0 - z) * n + z * h

    ctx = attention(af, nbf, 0)
    h = gru(ctx, af, gru_wih_ref, gru_whh_ref, gru_bih_ref, gru_bhh_ref, 0)
    act = jnp.maximum(h, 0.0)

    for d in range(1, RADIUS):
        act_hi, act_lo = _split2(act)
        nbg = gather(act_hi, act_lo, oh_a)
        ctx = attention(act, nbg, d)
        h = gru(ctx, h, gru_wih_ref, gru_whh_ref, gru_bih_ref,
                gru_bhh_ref, 3 * d)
        act = jnp.maximum(h, 0.0)

    amrow = jnp.concatenate([am3[m] for m in range(MB)], axis=1)
    masked = act * amrow
    mol = jnp.concatenate(
        [jnp.sum(masked[:, msl(m)], axis=1, keepdims=True)
         for m in range(MB)], axis=1)
    act_mol = jnp.maximum(mol, 0.0)
    msmrow = jnp.where(amrow == 0.0, _NEG, 0.0).astype(f32)
    msm3 = jnp.stack([msmrow[:, msl(m)] for m in range(MB)])
    am3s = jnp.stack([amrow[:, msl(m)] for m in range(MB)])

    w1m = w_malign_ref[0:1, :]
    w2m = w_malign_ref[1:2, :]
    bmal = b_malign_ref[0, 0]
    s_atomrow = _hdot(w2m, act)
    s_atom3 = jnp.stack([s_atomrow[:, msl(m)] for m in range(MB)])
    at = _dot(w_mattend_ref[:], act) + b_mattend_ref[:]

    for _ in range(T):
        s_molrow = _hdot(w1m, act_mol)
        ms3 = jnp.stack(
            [_leaky(s_molrow[:, m:m + 1] + s_atom3[m] + bmal) + msm3[m]
             for m in range(MB)])
        mx = jnp.max(ms3, axis=2, keepdims=True)
        e = jnp.exp(ms3 - mx)
        z = jnp.sum(e, axis=2, keepdims=True)
        maw3 = (e / z) * am3s
        mc = jnp.concatenate(
            [jnp.sum(at[:, msl(m)] * maw3[m], axis=1, keepdims=True)
             for m in range(MB)], axis=1)
        mc = _elu(mc)
        mol = gru(mc, mol, mgru_wih_ref, mgru_whh_ref,
                  mgru_bih_ref, mgru_bhh_ref, 0)
        act_mol = jnp.maximum(mol, 0.0)

    d_val = float(RADIUS - 2) if RADIUS > 1 else 0.0
    m1 = (_dot(w_m1_ref[:], mol) + _dot(w_m2_ref[:], mol + d_val)
          + b_metric_ref[:])
    outrow = _hdot(w_out_ref[:], m1) + b_out_ref[:]
    out_ref[:] = outrow.reshape(1, 1, MB)


def kernel(atom_list, bond_list, atom_degree_list, bond_degree_list,
           atom_mask, W_atom, b_atom, W_nb, b_nb, gru_wih, gru_whh,
           gru_bih, gru_bhh, W_align, b_align, W_attend, b_attend,
           mgru_wih, mgru_whh, mgru_bih, mgru_bhh, W_malign, b_malign,
           W_mattend, b_mattend, W_metric, b_metric, W_out, b_out):
    B = atom_list.shape[0]
    f32 = jnp.float32

    atom_t = jnp.transpose(atom_list, (2, 0, 1)).reshape(IN_FEAT, B * L)
    bond_t = jnp.transpose(bond_list, (2, 0, 1)).reshape(BOND_DIM, B * L)
    adl = jnp.transpose(atom_degree_list, (0, 2, 1)).reshape(B, 1, NBL)
    bdl = jnp.transpose(bond_degree_list, (0, 2, 1)).reshape(B, 1, NBL)
    adl = adl.astype(jnp.int32)
    bdl = bdl.astype(jnp.int32)
    amask = atom_mask.reshape(B, 1, L).astype(f32)

    gru_wih2 = gru_wih.reshape(RADIUS * 3, FP, FP)
    gru_whh2 = gru_whh.reshape(RADIUS * 3, FP, FP)
    gru_bih2 = gru_bih.reshape(RADIUS * 3, FP, 1)
    gru_bhh2 = gru_bhh.reshape(RADIUS * 3, FP, 1)
    w_align2 = W_align.reshape(RADIUS * 2, FP)
    b_align2 = b_align.reshape(RADIUS, 1)
    b_attend2 = b_attend.reshape(RADIUS, FP, 1)
    mgru_wih2 = mgru_wih.reshape(3, FP, FP)
    mgru_whh2 = mgru_whh.reshape(3, FP, FP)
    mgru_bih2 = mgru_bih.reshape(3, FP, 1)
    mgru_bhh2 = mgru_bhh.reshape(3, FP, 1)
    w_malign2 = W_malign.reshape(2, FP)
    b_malign2 = b_malign.reshape(1, 1)
    b_mattend2 = b_mattend.reshape(FP, 1)
    w_m1 = W_metric[:, :FP]
    w_m2 = W_metric[:, FP:]
    b_metric2 = b_metric.reshape(FP, 1)
    b_out2 = b_out.reshape(1, 1)
    b_atom2 = b_atom.reshape(FP, 1)
    b_nb2 = b_nb.reshape(FP, 1)
    w_nb_a = W_nb[:, :IN_FEAT]
    w_nb_b = W_nb[:, IN_FEAT:]

    grid = (B // MB,)

    args = (
        atom_t, bond_t, adl, bdl, amask,
        W_atom, b_atom2, w_nb_a, w_nb_b, b_nb2,
        gru_wih2, gru_whh2, gru_bih2, gru_bhh2,
        w_align2, b_align2, W_attend, b_attend2,
        mgru_wih2, mgru_whh2, mgru_bih2, mgru_bhh2,
        w_malign2, b_malign2, W_mattend, b_mattend2,
        w_m1, w_m2, b_metric2, W_out, b_out2,
    )

    def wspec(shape):
        nd = len(shape)
        return pl.BlockSpec(shape, lambda i, nd=nd: (0,) * nd)

    in_specs = [
        pl.BlockSpec((IN_FEAT, MB * L), lambda i: (0, i)),
        pl.BlockSpec((BOND_DIM, MB * L), lambda i: (0, i)),
        pl.BlockSpec((MB, 1, NBL), lambda i: (i, 0, 0)),
        pl.BlockSpec((MB, 1, NBL), lambda i: (i, 0, 0)),
        pl.BlockSpec((MB, 1, L), lambda i: (i, 0, 0)),
    ] + [wspec(a.shape) for a in args[5:]]

    out = pl.pallas_call(
        _fused_kernel,
        grid=grid,
        in_specs=in_specs,
        out_specs=pl.BlockSpec((1, 1, MB), lambda i: (i, 0, 0)),
        out_shape=jax.ShapeDtypeStruct((B // MB, 1, MB), f32),
        compiler_params=pltpu.CompilerParams(
            dimension_semantics=("arbitrary",),
        ),
    )(*args)
    return out.reshape(B, 1)


# speedup vs baseline: 2.2337x; 1.1293x over previous
import jax
import jax.numpy as jnp
from jax.experimental import pallas as pl
from jax.experimental.pallas import tpu as pltpu

RADIUS = 3
T = 2
IN_FEAT = 39
BOND_DIM = 10
FP = 64
L = 64
NB = 6
NBL = NB * L
MB = 16

_NEG = -9e8
_SLOPE = 0.01


def _leaky(x):
    return jnp.where(x > 0, x, _SLOPE * x)


def _elu(x):
    return jnp.where(x > 0, x, jnp.exp(jnp.minimum(x, 0.0)) - 1.0)


def _dot(a, b):
    return jnp.dot(a, b, preferred_element_type=jnp.float32)


def _hdot(a, b):
    return jnp.dot(a, b, preferred_element_type=jnp.float32)


def _xdot(a, b):
    return jnp.dot(a, b, preferred_element_type=jnp.float32,
                   precision=jax.lax.Precision.HIGHEST)


def _split2(x):
    hi = x.astype(jnp.bfloat16).astype(jnp.float32)
    return hi, x - hi


def _fused_kernel(
    atom_ref, bond_ref, adl_ref, bdl_ref, amask_ref,
    w_atom_ref, b_atom_ref, w_nb_a_ref, w_nb_b_ref, b_nb_ref,
    gru_wih_ref, gru_whh_ref, gru_bih_ref, gru_bhh_ref,
    w_align_ref, b_align_ref, w_attend_ref, b_attend_ref,
    mgru_wih_ref, mgru_whh_ref, mgru_bih_ref, mgru_bhh_ref,
    w_malign_ref, b_malign_ref, w_mattend_ref, b_mattend_ref,
    w_m1_ref, w_m2_ref, b_metric_ref, w_out_ref, b_out_ref,
    out_ref,
):
    f32 = jnp.float32
    xa = atom_ref[:]
    xb = bond_ref[:]
    adl3 = adl_ref[:]
    bdl3 = bdl_ref[:]
    am3 = amask_ref[:]

    iota_s = jax.lax.broadcasted_iota(jnp.int32, (L, NBL), 0)
    oh_a = [(adl3[m] == iota_s).astype(f32) for m in range(MB)]
    oh_b = [(bdl3[m] == iota_s).astype(f32) for m in range(MB)]

    attend3 = (adl3 != L - 1).astype(f32)
    smask3 = jnp.where(adl3 == L - 1, _NEG, 0.0).astype(f32)

    def msl(m):
        return slice(m * L, (m + 1) * L)

    def jsl(j):
        return slice(j * L, (j + 1) * L)

    def gather(hi, lo, oh):
        return jnp.concatenate(
            [_dot(hi[:, msl(m)], oh[m]) + _dot(lo[:, msl(m)], oh[m])
             for m in range(MB)], axis=1)

    af = _leaky(_dot(w_atom_ref[:], xa) + b_atom_ref[:])

    xa_hi, xa_lo = _split2(xa)
    xb_hi, xb_lo = _split2(xb)
    ga = gather(xa_hi, xa_lo, oh_a)
    gb = gather(xb_hi, xb_lo, oh_b)
    nbf = _leaky(_dot(w_nb_a_ref[:], ga) + _dot(w_nb_b_ref[:], gb)
                 + b_nb_ref[:])

    def attention(act, nb, d):
        w1 = w_align_ref[2 * d:2 * d + 1, :]
        w2 = w_align_ref[2 * d + 1:2 * d + 2, :]
        bal = b_align_ref[d:d + 1, :].reshape(1, 1, 1)
        s2row = _hdot(w2, nb)
        s23 = jnp.stack(
            [s2row[:, m * NBL:(m + 1) * NBL] for m in range(MB)])
        s1row = _hdot(w1, act)
        s13 = jnp.stack([s1row[:, msl(m)] for m in range(MB)])
        nt = _dot(w_attend_ref[d], nb) + b_attend_ref[d]
        sj = [
            _leaky(s13 + s23[:, :, jsl(j)] + bal) + smask3[:, :, jsl(j)]
            for j in range(NB)
        ]
        mx = sj[0]
        for s in sj[1:]:
            mx = jnp.maximum(mx, s)
        ej = [jnp.exp(s - mx) for s in sj]
        z = ej[0]
        for e in ej[1:]:
            z = z + e
        aw3 = jnp.concatenate(
            [(ej[j] / z) * attend3[:, :, jsl(j)] for j in range(NB)],
            axis=2)
        ctxs = []
        for m in range(MB):
            ws = nt[:, m * NBL:(m + 1) * NBL] * aw3[m]
            acc = ws[:, jsl(0)]
            for j in range(1, NB):
                acc = acc + ws[:, jsl(j)]
            ctxs.append(acc)
        return _elu(jnp.concatenate(ctxs, axis=1))

    def gru(x, h, wih_ref, whh_ref, bih_ref, bhh_ref, base):
        gi_r = _dot(wih_ref[base + 0], x) + bih_ref[base + 0]
        gi_z = _dot(wih_ref[base + 1], x) + bih_ref[base + 1]
        gi_n = _dot(wih_ref[base + 2], x) + bih_ref[base + 2]
        gh_r = _dot(whh_ref[base + 0], h) + bhh_ref[base + 0]
        gh_z = _dot(whh_ref[base + 1], h) + bhh_ref[base + 1]
        gh_n = _dot(whh_ref[base + 2], h) + bhh_ref[base + 2]
        r = jax.nn.sigmoid(gi_r + gh_r)
        z = jax.nn.sigmoid(gi_z + gh_z)
        n = jnp.tanh(gi_n + r * gh_n)
        return (1.0 - z) * n + z * h

    ctx = attention(af, nbf, 0)
    h = gru(ctx, af, gru_wih_ref, gru_whh_ref, gru_bih_ref, gru_bhh_ref, 0)
    act = jnp.maximum(h, 0.0)

    for d in range(1, RADIUS):
        act_hi, act_lo = _split2(act)
        nbg = gather(act_hi, act_lo, oh_a)
        ctx = attention(act, nbg, d)
        h = gru(ctx, h, gru_wih_ref, gru_whh_ref, gru_bih_ref,
                gru_bhh_ref, 3 * d)
        act = jnp.maximum(h, 0.0)

    amrow = jnp.concatenate([am3[m] for m in range(MB)], axis=1)
    masked = act * amrow
    mol = jnp.concatenate(
        [jnp.sum(masked[:, msl(m)], axis=1, keepdims=True)
         for m in range(MB)], axis=1)
    act_mol = jnp.maximum(mol, 0.0)
    msmrow = jnp.where(amrow == 0.0, _NEG, 0.0).astype(f32)
    msm3 = jnp.stack([msmrow[:, msl(m)] for m in range(MB)])
    am3s = jnp.stack([amrow[:, msl(m)] for m in range(MB)])

    w1m = w_malign_ref[0:1, :]
    w2m = w_malign_ref[1:2, :]
    bmal = b_malign_ref[0, 0]
    s_atomrow = _xdot(w2m, act)
    s_atom3 = jnp.stack([s_atomrow[:, msl(m)] for m in range(MB)])
    at = _dot(w_mattend_ref[:], act) + b_mattend_ref[:]

    for _ in range(T):
        s_molrow = _xdot(w1m, act_mol)
        ms3 = jnp.stack(
            [_leaky(s_molrow[:, m:m + 1] + s_atom3[m] + bmal) + msm3[m]
             for m in range(MB)])
        mx = jnp.max(ms3, axis=2, keepdims=True)
        e = jnp.exp(ms3 - mx)
        z = jnp.sum(e, axis=2, keepdims=True)
        maw3 = (e / z) * am3s
        mc = jnp.concatenate(
            [jnp.sum(at[:, msl(m)] * maw3[m], axis=1, keepdims=True)
             for m in range(MB)], axis=1)
        mc = _elu(mc)
        mol = gru(mc, mol, mgru_wih_ref, mgru_whh_ref,
                  mgru_bih_ref, mgru_bhh_ref, 0)
        act_mol = jnp.maximum(mol, 0.0)

    d_val = float(RADIUS - 2) if RADIUS > 1 else 0.0
    m1 = (_xdot(w_m1_ref[:], mol) + _xdot(w_m2_ref[:], mol + d_val)
          + b_metric_ref[:])
    outrow = _xdot(w_out_ref[:], m1) + b_out_ref[:]
    out_ref[:] = outrow.reshape(1, 1, MB)


def kernel(atom_list, bond_list, atom_degree_list, bond_degree_list,
           atom_mask, W_atom, b_atom, W_nb, b_nb, gru_wih, gru_whh,
           gru_bih, gru_bhh, W_align, b_align, W_attend, b_attend,
           mgru_wih, mgru_whh, mgru_bih, mgru_bhh, W_malign, b_malign,
           W_mattend, b_mattend, W_metric, b_metric, W_out, b_out):
    B = atom_list.shape[0]
    f32 = jnp.float32

    atom_t = jnp.transpose(atom_list, (2, 0, 1)).reshape(IN_FEAT, B * L)
    bond_t = jnp.transpose(bond_list, (2, 0, 1)).reshape(BOND_DIM, B * L)
    adl = jnp.transpose(atom_degree_list, (0, 2, 1)).reshape(B, 1, NBL)
    bdl = jnp.transpose(bond_degree_list, (0, 2, 1)).reshape(B, 1, NBL)
    adl = adl.astype(jnp.int32)
    bdl = bdl.astype(jnp.int32)
    amask = atom_mask.reshape(B, 1, L).astype(f32)

    gru_wih2 = gru_wih.reshape(RADIUS * 3, FP, FP)
    gru_whh2 = gru_whh.reshape(RADIUS * 3, FP, FP)
    gru_bih2 = gru_bih.reshape(RADIUS * 3, FP, 1)
    gru_bhh2 = gru_bhh.reshape(RADIUS * 3, FP, 1)
    w_align2 = W_align.reshape(RADIUS * 2, FP)
    b_align2 = b_align.reshape(RADIUS, 1)
    b_attend2 = b_attend.reshape(RADIUS, FP, 1)
    mgru_wih2 = mgru_wih.reshape(3, FP, FP)
    mgru_whh2 = mgru_whh.reshape(3, FP, FP)
    mgru_bih2 = mgru_bih.reshape(3, FP, 1)
    mgru_bhh2 = mgru_bhh.reshape(3, FP, 1)
    w_malign2 = W_malign.reshape(2, FP)
    b_malign2 = b_malign.reshape(1, 1)
    b_mattend2 = b_mattend.reshape(FP, 1)
    w_m1 = W_metric[:, :FP]
    w_m2 = W_metric[:, FP:]
    b_metric2 = b_metric.reshape(FP, 1)
    b_out2 = b_out.reshape(1, 1)
    b_atom2 = b_atom.reshape(FP, 1)
    b_nb2 = b_nb.reshape(FP, 1)
    w_nb_a = W_nb[:, :IN_FEAT]
    w_nb_b = W_nb[:, IN_FEAT:]

    grid = (B // MB,)

    args = (
        atom_t, bond_t, adl, bdl, amask,
        W_atom, b_atom2, w_nb_a, w_nb_b, b_nb2,
        gru_wih2, gru_whh2, gru_bih2, gru_bhh2,
        w_align2, b_align2, W_attend, b_attend2,
        mgru_wih2, mgru_whh2, mgru_bih2, mgru_bhh2,
        w_malign2, b_malign2, W_mattend, b_mattend2,
        w_m1, w_m2, b_metric2, W_out, b_out2,
    )

    def wspec(shape):
        nd = len(shape)
        return pl.BlockSpec(shape, lambda i, nd=nd: (0,) * nd)

    in_specs = [
        pl.BlockSpec((IN_FEAT, MB * L), lambda i: (0, i)),
        pl.BlockSpec((BOND_DIM, MB * L), lambda i: (0, i)),
        pl.BlockSpec((MB, 1, NBL), lambda i: (i, 0, 0)),
        pl.BlockSpec((MB, 1, NBL), lambda i: (i, 0, 0)),
        pl.BlockSpec((MB, 1, L), lambda i: (i, 0, 0)),
    ] + [wspec(a.shape) for a in args[5:]]

    out = pl.pallas_call(
        _fused_kernel,
        grid=grid,
        in_specs=in_specs,
        out_specs=pl.BlockSpec((1, 1, MB), lambda i: (i, 0, 0)),
        out_shape=jax.ShapeDtypeStruct((B // MB, 1, MB), f32),
        compiler_params=pltpu.CompilerParams(
            dimension_semantics=("arbitrary",),
        ),
    )(*args)
    return out.reshape(B, 1)
